# baseline (speedup 1.0000x reference)
import functools

import jax
import jax.numpy as jnp
from jax import lax
from jax.experimental import pallas as pl
from jax.experimental.pallas import tpu as pltpu
from jax.experimental.pallas import tpu_sc as plsc

D = 64
SEQ = 200
BATCH = 4096
NC, NS = 2, 16
NW = NC * NS
BB = BATCH // NW
SCALE = 8.0


def _body(xt_hbm, tab_hbm, pos_hbm, out_hbm, xv, idx2, slab, yb, pos_v, sem):
    wid = lax.axis_index("s") * NC + lax.axis_index("c")
    b0 = wid * BB
    pltpu.sync_copy(pos_hbm, pos_v)
    iota = lax.broadcasted_iota(jnp.int32, (16,), 0)

    def seq_body(s, carry):
        pltpu.sync_copy(xt_hbm.at[s, pl.ds(b0, BB)], xv)
        for k in range(BB // 16):
            sl = pl.ds(k * 16, 16)
            idx2[sl] = lax.shift_right_logical(xv[sl], 1)
        pltpu.async_copy(tab_hbm.at[idx2], slab, sem).wait()

        rows = []
        cols = []
        for g in range(BB // 16):
            vv = xv[pl.ds(g * 16, 16)]
            rows.append(iota + (g * 16))
            cols.append((vv & 1) * 64)
        zero = jnp.zeros((16,), jnp.int32)
        for d in range(D):
            p = plsc.load_gather(pos_v, [zero + (s * D + d)])
            for g in range(BB // 16):
                v = plsc.load_gather(slab, [rows[g], cols[g] + d])
                yb[d, pl.ds(g * 16, 16)] = v * SCALE + p
        pltpu.sync_copy(yb, out_hbm.at[s, :, pl.ds(b0, BB)])
        return carry

    lax.fori_loop(0, SEQ, seq_body, 0)


@jax.jit
def kernel(x, token_table, pos_table):
    xt = x.T.astype(jnp.int32)
    tab2 = token_table.reshape(500000, 128)
    pos_flat = pos_table.reshape(-1)
    mesh = plsc.VectorSubcoreMesh(core_axis_name="c", subcore_axis_name="s")
    run = pl.kernel(
        _body,
        mesh=mesh,
        compiler_params=pltpu.CompilerParams(
            use_tc_tiling_on_sc=True, needs_layout_passes=False
        ),
        out_type=jax.ShapeDtypeStruct((SEQ, D, BATCH), jnp.float32),
        scratch_types=[
            pltpu.VMEM((BB,), jnp.int32),
            pltpu.VMEM((BB,), jnp.int32),
            pltpu.VMEM((BB, 128), jnp.float32),
            pltpu.VMEM((D, BB), jnp.float32),
            pltpu.VMEM((SEQ * D,), jnp.float32),
            pltpu.SemaphoreType.DMA,
        ],
    )
    y = run(xt, tab2, pos_flat)
    return y.transpose(2, 0, 1)


# speedup vs baseline: 1.8770x; 1.8770x over previous
import functools

import jax
import jax.numpy as jnp
from jax import lax
from jax.experimental import pallas as pl
from jax.experimental.pallas import tpu as pltpu
from jax.experimental.pallas import tpu_sc as plsc

D = 64
SEQ = 200
BATCH = 4096
NC, NS = 2, 16
NW = NC * NS
BB = BATCH // NW
NG = BB // 16
SCALE = 8.0


def _body(xt_hbm, tab_hbm, pos_hbm, out_hbm, xall, idx2a, idx2b,
          slaba, slabb, yb, pos_v, sema, semb):
    wid = lax.axis_index("s") * NC + lax.axis_index("c")
    b0 = wid * BB
    pltpu.sync_copy(pos_hbm, pos_v)
    pltpu.sync_copy(xt_hbm.at[:, pl.ds(b0, BB)], xall)
    iota = lax.broadcasted_iota(jnp.int32, (16,), 0)
    zero = jnp.zeros((16,), jnp.int32)

    def prep_idx(s, idx2):
        for k in range(NG):
            sl = pl.ds(k * 16, 16)
            idx2[sl] = lax.shift_right_logical(xall[s, sl], 1)

    def compute(s, slab):
        rows = []
        cols = []
        for g in range(NG):
            vv = xall[s, pl.ds(g * 16, 16)]
            rows.append(iota + (g * 16))
            cols.append((vv & 1) * 64)

        @plsc.parallel_loop(0, D, unroll=4)
        def dloop(d):
            p = plsc.load_gather(pos_v, [zero + (s * D + d)])
            for g in range(NG):
                v = plsc.load_gather(slab, [rows[g], cols[g] + d])
                yb[d, pl.ds(g * 16, 16)] = v * SCALE + p

        pltpu.sync_copy(yb, out_hbm.at[s, :, pl.ds(b0, BB)])

    prep_idx(0, idx2a)
    pltpu.async_copy(tab_hbm.at[idx2a], slaba, sema)

    def pair_body(c, carry):
        s0 = 2 * c
        prep_idx(s0 + 1, idx2b)
        pltpu.async_copy(tab_hbm.at[idx2b], slabb, semb)
        pltpu.make_async_copy(tab_hbm.at[idx2a], slaba, sema).wait()
        compute(s0, slaba)
        @pl.when(c < SEQ // 2 - 1)
        def _():
            prep_idx(s0 + 2, idx2a)
            pltpu.async_copy(tab_hbm.at[idx2a], slaba, sema)

        pltpu.make_async_copy(tab_hbm.at[idx2b], slabb, semb).wait()
        compute(s0 + 1, slabb)
        return carry

    lax.fori_loop(0, SEQ // 2, pair_body, 0)


@jax.jit
def kernel(x, token_table, pos_table):
    xt = x.T.astype(jnp.int32)
    tab2 = token_table.reshape(500000, 128)
    pos_flat = pos_table.reshape(-1)
    mesh = plsc.VectorSubcoreMesh(core_axis_name="c", subcore_axis_name="s")
    run = pl.kernel(
        _body,
        mesh=mesh,
        compiler_params=pltpu.CompilerParams(
            use_tc_tiling_on_sc=True, needs_layout_passes=False
        ),
        out_type=jax.ShapeDtypeStruct((SEQ, D, BATCH), jnp.float32),
        scratch_types=[
            pltpu.VMEM((SEQ, BB), jnp.int32),
            pltpu.VMEM((BB,), jnp.int32),
            pltpu.VMEM((BB,), jnp.int32),
            pltpu.VMEM((BB, 128), jnp.float32),
            pltpu.VMEM((BB, 128), jnp.float32),
            pltpu.VMEM((D, BB), jnp.float32),
            pltpu.VMEM((SEQ * D,), jnp.float32),
            pltpu.SemaphoreType.DMA,
            pltpu.SemaphoreType.DMA,
        ],
    )
    y = run(xt, tab2, pos_flat)
    return y.transpose(2, 0, 1)


# speedup vs baseline: 1.9805x; 1.0552x over previous
import functools

import jax
import jax.numpy as jnp
from jax import lax
from jax.experimental import pallas as pl
from jax.experimental.pallas import tpu as pltpu
from jax.experimental.pallas import tpu_sc as plsc

D = 64
SEQ = 200
BATCH = 4096
NC, NS = 2, 16
NW = NC * NS
BB = BATCH // NW
NG = BB // 16
SCALE = 8.0


def _body(xt_hbm, tab_hbm, pos_hbm, out_hbm, xall, idx2a, idx2b,
          slaba, slabb, yba, ybb, pos_v, sema, semb, semwa, semwb):
    wid = lax.axis_index("s") * NC + lax.axis_index("c")
    b0 = wid * BB
    pltpu.sync_copy(pos_hbm, pos_v)
    pltpu.sync_copy(xt_hbm.at[:, pl.ds(b0, BB)], xall)
    iota = lax.broadcasted_iota(jnp.int32, (16,), 0)
    zero = jnp.zeros((16,), jnp.int32)

    def prep_idx(s, idx2):
        for k in range(NG):
            sl = pl.ds(k * 16, 16)
            idx2[sl] = lax.shift_right_logical(xall[s, sl], 1)

    def compute(s, slab, yb, sem):
        rows = []
        cols = []
        for g in range(NG):
            vv = xall[s, pl.ds(g * 16, 16)]
            rows.append(iota + (g * 16))
            cols.append((vv & 1) * 64)

        @plsc.parallel_loop(0, D, unroll=8)
        def dloop(d):
            p = plsc.load_gather(pos_v, [zero + (s * D + d)])
            for g in range(NG):
                v = plsc.load_gather(slab, [rows[g], cols[g] + d])
                yb[d, pl.ds(g * 16, 16)] = v * SCALE + p

        pltpu.async_copy(yb, out_hbm.at[s, :, pl.ds(b0, BB)], sem)

    prep_idx(0, idx2a)
    pltpu.async_copy(tab_hbm.at[idx2a], slaba, sema)

    def drain_write(yb, sem, s):
        pltpu.make_async_copy(yb, out_hbm.at[s, :, pl.ds(b0, BB)], sem).wait()

    def pair_body(c, carry):
        s0 = 2 * c
        prep_idx(s0 + 1, idx2b)
        pltpu.async_copy(tab_hbm.at[idx2b], slabb, semb)
        pltpu.make_async_copy(tab_hbm.at[idx2a], slaba, sema).wait()

        @pl.when(c > 0)
        def _():
            drain_write(yba, semwa, s0 - 2)

        compute(s0, slaba, yba, semwa)
        @pl.when(c < SEQ // 2 - 1)
        def _():
            prep_idx(s0 + 2, idx2a)
            pltpu.async_copy(tab_hbm.at[idx2a], slaba, sema)

        pltpu.make_async_copy(tab_hbm.at[idx2b], slabb, semb).wait()

        @pl.when(c > 0)
        def _():
            drain_write(ybb, semwb, s0 - 1)

        compute(s0 + 1, slabb, ybb, semwb)
        return carry

    lax.fori_loop(0, SEQ // 2, pair_body, 0)
    drain_write(yba, semwa, SEQ - 2)
    drain_write(ybb, semwb, SEQ - 1)


@jax.jit
def kernel(x, token_table, pos_table):
    xt = x.T.astype(jnp.int32)
    tab2 = token_table.reshape(500000, 128)
    pos_flat = pos_table.reshape(-1)
    mesh = plsc.VectorSubcoreMesh(core_axis_name="c", subcore_axis_name="s")
    run = pl.kernel(
        _body,
        mesh=mesh,
        compiler_params=pltpu.CompilerParams(
            use_tc_tiling_on_sc=True, needs_layout_passes=False
        ),
        out_type=jax.ShapeDtypeStruct((SEQ, D, BATCH), jnp.float32),
        scratch_types=[
            pltpu.VMEM((SEQ, BB), jnp.int32),
            pltpu.VMEM((BB,), jnp.int32),
            pltpu.VMEM((BB,), jnp.int32),
            pltpu.VMEM((BB, 128), jnp.float32),
            pltpu.VMEM((BB, 128), jnp.float32),
            pltpu.VMEM((D, BB), jnp.float32),
            pltpu.VMEM((D, BB), jnp.float32),
            pltpu.VMEM((SEQ * D,), jnp.float32),
            pltpu.SemaphoreType.DMA,
            pltpu.SemaphoreType.DMA,
            pltpu.SemaphoreType.DMA,
            pltpu.SemaphoreType.DMA,
        ],
    )
    y = run(xt, tab2, pos_flat)
    return y.transpose(2, 0, 1)
